# Initial kernel scaffold; baseline (speedup 1.0000x reference)
import jax
import jax.numpy as jnp
from jax.experimental import pallas as pl


def kernel(x, edge_index, W1, b1, W2, b2, Wf, bf):
    raise NotImplementedError("write your pallas kernel here")


# speedup vs baseline: 7.7513x; 7.7513x over previous
import functools

import jax
import jax.numpy as jnp
from jax import lax
from jax.experimental import pallas as pl
from jax.experimental.pallas import tpu as pltpu
from jax.experimental.pallas import tpu_sc as plsc

N = 100000
E = 1600000
F_IN = 128
H = 16

NC = 2
NS = 16
NW = NC * NS

CHUNK = 128
NCHT = 12800
EPAD = NCHT * CHUNK
NCH = NCHT // NS
SEG = 160
NSTG = NCH // SEG
SEGE = SEG * CHUNK
DUMP = SEGE + CHUNK

NPAD = 102400
HALF = NPAD // 2
ACC = HALF + 128
TNR = HALF // NS
ROUNDS = 5
RR = NPAD // (NC * ROUNDS)
QACC = RR + 128
QNR = RR // NS
CZ = 800


TRASH_W = 128


def _remap(dst_v, nrows, base, size, trash):
    lane = lax.iota(jnp.int32, 16)

    def row(j, carry):
        for k in range(CHUNK // 16):
            d = dst_v[j, pl.ds(16 * k, 16)]
            loc = d - base
            ok = (loc >= 0) & (loc < size)
            dst_v[j, pl.ds(16 * k, 16)] = jnp.where(
                ok, loc, trash + 16 * k + lane)
        return carry

    lax.fori_loop(0, nrows, row, 0)


def _deg_body(dst_hbm, out_hbm, dst_v, ones_v, stage_v, sh_deg, sem):
    c = lax.axis_index("c")
    s = lax.axis_index("s")
    base = c * HALF
    pltpu.sync_copy(dst_hbm.at[pl.ds(s * NCH, NCH)], dst_v)
    for i in range(CHUNK // 16):
        ones_v[pl.ds(16 * i, 16)] = jnp.ones((16,), jnp.float32)

    def zrow(i, carry):
        stage_v[pl.ds(16 * i, 16)] = jnp.zeros((16,), jnp.float32)
        return carry

    lax.fori_loop(0, TNR // 16, zrow, 0)
    _remap(dst_v, NCH, base, HALF, HALF)
    pltpu.sync_copy(stage_v, sh_deg.at[pl.ds(s * TNR, TNR)])
    plsc.subcore_barrier()

    def grp(p, carry):
        for b in range(8):
            pltpu.async_copy(ones_v, sh_deg.at[dst_v.at[8 * p + b]], sem, add=True)
        for b in range(8):
            pltpu.make_async_copy(ones_v, sh_deg.at[dst_v.at[8 * p + b]], sem).wait()
        return carry

    lax.fori_loop(0, NCH // 8, grp, 0)
    plsc.subcore_barrier()
    pltpu.sync_copy(sh_deg.at[pl.ds(s * TNR, TNR)], stage_v)
    pltpu.sync_copy(stage_v, out_hbm.at[pl.ds(base + s * TNR, TNR)])


def _sc_degree(dst2d):
    mesh = plsc.VectorSubcoreMesh(core_axis_name="c", subcore_axis_name="s")
    return pl.kernel(
        _deg_body,
        out_type=jax.ShapeDtypeStruct((NPAD,), jnp.float32),
        mesh=mesh,
        scratch_types=[
            pltpu.VMEM((NCH, CHUNK), jnp.int32),
            pltpu.VMEM((CHUNK,), jnp.float32),
            pltpu.VMEM((TNR,), jnp.float32),
            pltpu.VMEM_SHARED((ACC,), jnp.float32),
            pltpu.SemaphoreType.DMA,
        ],
        compiler_params=pltpu.CompilerParams(
            use_tc_tiling_on_sc=False, internal_scratch_in_bytes=65536),
    )(dst2d)


def _agg_body(g_hbm, src_hbm, dst_hbm, out_hbm,
              src_v, dst_v, rows_v, stage_v, sh_acc, semg, sems):
    c = lax.axis_index("c")
    s = lax.axis_index("s")

    def zrow(i, carry):
        stage_v[i] = jnp.zeros((H,), jnp.float32)
        return carry

    for r in range(ROUNDS):
        base = (r * NC + c) * RR
        lax.fori_loop(0, QNR, zrow, 0)
        pltpu.sync_copy(stage_v, sh_acc.at[pl.ds(s * QNR, QNR)])
        plsc.subcore_barrier()

        for g in range(NSTG):
            cb = s * NCH + g * SEG
            pltpu.sync_copy(src_hbm.at[pl.ds(cb, SEG)], src_v)
            pltpu.sync_copy(dst_hbm.at[pl.ds(cb, SEG)], dst_v)
            _remap(dst_v, SEG, base, RR, RR)

            BLK = 16
            nblk = SEG // BLK

            def fire_gathers(b, m):
                for q in range(BLK):
                    pltpu.async_copy(
                        g_hbm.at[src_v.at[BLK * b + q]],
                        rows_v.at[m, pl.ds(q * CHUNK, CHUNK)], semg)

            fire_gathers(jnp.int32(0), jnp.int32(0))

            def blk(b, carry):
                m = lax.rem(b, 2)
                pltpu.make_async_copy(
                    g_hbm.at[pl.ds(0, BLK * CHUNK)], rows_v.at[m], semg).wait()
                for q in range(BLK):
                    pltpu.async_copy(
                        rows_v.at[m, pl.ds(q * CHUNK, CHUNK)],
                        sh_acc.at[dst_v.at[BLK * b + q]], sems, add=True)

                @pl.when(b >= 1)
                def _():
                    pltpu.make_async_copy(
                        rows_v.at[1 - m], sh_acc.at[pl.ds(0, BLK * CHUNK)],
                        sems).wait()

                @pl.when(b < nblk - 1)
                def _():
                    fire_gathers(b + 1, 1 - m)

                return carry

            lax.fori_loop(0, nblk, blk, 0)
            pltpu.make_async_copy(
                rows_v.at[lax.rem(jnp.int32(nblk - 1), 2)],
                sh_acc.at[pl.ds(0, BLK * CHUNK)], sems).wait()

        plsc.subcore_barrier()
        pltpu.sync_copy(sh_acc.at[pl.ds(s * QNR, QNR)], stage_v)
        pltpu.sync_copy(stage_v, out_hbm.at[pl.ds(base + s * QNR, QNR)])
        plsc.subcore_barrier()


def _sc_aggregate(g, src2d, dst2d):
    mesh = plsc.VectorSubcoreMesh(core_axis_name="c", subcore_axis_name="s")
    return pl.kernel(
        _agg_body,
        out_type=jax.ShapeDtypeStruct((NPAD, H), jnp.float32),
        mesh=mesh,
        scratch_types=[
            pltpu.VMEM((SEG, CHUNK), jnp.int32),
            pltpu.VMEM((SEG, CHUNK), jnp.int32),
            pltpu.VMEM((2, 16 * CHUNK, H), jnp.float32),
            pltpu.VMEM((QNR, H), jnp.float32),
            pltpu.VMEM_SHARED((QACC, H), jnp.float32),
            pltpu.SemaphoreType.DMA,
            pltpu.SemaphoreType.DMA,
        ],
        compiler_params=pltpu.CompilerParams(
            use_tc_tiling_on_sc=False, internal_scratch_in_bytes=65536),
    )(g, src2d, dst2d)


R = 2048
GRID = (N + R - 1) // R


def _tc1_body(x_ref, w1_ref, d_ref, g_ref, dis_ref):
    deg = d_ref[...] + 1.0
    dis = lax.rsqrt(deg)
    h = jnp.dot(x_ref[...], w1_ref[...], preferred_element_type=jnp.float32)
    g_ref[...] = h * dis[:, None]
    dis_ref[...] = dis


def _tc1(x, W1, deg):
    return pl.pallas_call(
        _tc1_body,
        grid=(GRID,),
        in_specs=[
            pl.BlockSpec((R, F_IN), lambda j: (j, 0)),
            pl.BlockSpec((F_IN, H), lambda j: (0, 0)),
            pl.BlockSpec((R,), lambda j: (j,)),
        ],
        out_specs=[
            pl.BlockSpec((R, H), lambda j: (j, 0)),
            pl.BlockSpec((R,), lambda j: (j,)),
        ],
        out_shape=[
            jax.ShapeDtypeStruct((N, H), jnp.float32),
            jax.ShapeDtypeStruct((N,), jnp.float32),
        ],
    )(x, W1, deg)


def _tc2_body(s_ref, g1_ref, dis_ref, b1_ref, w2_ref, g2_ref):
    dis = dis_ref[...][:, None]
    ssum = s_ref[...] + g1_ref[...]
    h2 = jnp.maximum(dis * ssum + b1_ref[...][None, :], 0.0)
    g2_ref[...] = jnp.dot(h2, w2_ref[...], preferred_element_type=jnp.float32) * dis


def _tc2(sagg, g1, dis, b1, W2):
    return pl.pallas_call(
        _tc2_body,
        grid=(GRID,),
        in_specs=[
            pl.BlockSpec((R, H), lambda j: (j, 0)),
            pl.BlockSpec((R, H), lambda j: (j, 0)),
            pl.BlockSpec((R,), lambda j: (j,)),
            pl.BlockSpec((H,), lambda j: (0,)),
            pl.BlockSpec((H, H), lambda j: (0, 0)),
        ],
        out_specs=pl.BlockSpec((R, H), lambda j: (j, 0)),
        out_shape=jax.ShapeDtypeStruct((N, H), jnp.float32),
    )(sagg, g1, dis, b1, W2)


def _tc3_body(s_ref, g2_ref, dis_ref, b2_ref, wf_ref, bf_ref, out_ref):
    dis = dis_ref[...][:, None]
    ssum = s_ref[...] + g2_ref[...]
    h3 = jnp.maximum(dis * ssum + b2_ref[...][None, :], 0.0)
    out_ref[...] = (jnp.dot(h3, wf_ref[...], preferred_element_type=jnp.float32)
                    + bf_ref[...][None, :])


def _tc3(sagg, g2, dis, b2, Wf, bf):
    return pl.pallas_call(
        _tc3_body,
        grid=(GRID,),
        in_specs=[
            pl.BlockSpec((R, H), lambda j: (j, 0)),
            pl.BlockSpec((R, H), lambda j: (j, 0)),
            pl.BlockSpec((R,), lambda j: (j,)),
            pl.BlockSpec((H,), lambda j: (0,)),
            pl.BlockSpec((H, H), lambda j: (0, 0)),
            pl.BlockSpec((H,), lambda j: (0,)),
        ],
        out_specs=pl.BlockSpec((R, H), lambda j: (j, 0)),
        out_shape=jax.ShapeDtypeStruct((N, H), jnp.float32),
    )(sagg, g2, dis, b2, Wf, bf)


def kernel(x, edge_index, W1, b1, W2, b2, Wf, bf):
    src = edge_index[0]
    dst = edge_index[1]
    npad_e = EPAD - E
    src2d = jnp.concatenate(
        [src, jnp.zeros((npad_e,), jnp.int32)]).reshape(NCHT, CHUNK)
    dst2d = jnp.concatenate(
        [dst, jnp.full((npad_e,), N, jnp.int32)]).reshape(NCHT, CHUNK)

    deg = _sc_degree(dst2d)
    g1, dis = _tc1(x, W1, deg)
    s1 = _sc_aggregate(g1, src2d, dst2d)
    g2 = _tc2(s1, g1, dis, b1, W2)
    s2 = _sc_aggregate(g2, src2d, dst2d)
    return _tc3(s2, g2, dis, b2, Wf, bf)


# speedup vs baseline: 7.9916x; 1.0310x over previous
import jax
import jax.numpy as jnp
from jax import lax
from jax.experimental import pallas as pl
from jax.experimental.pallas import tpu as pltpu
from jax.experimental.pallas import tpu_sc as plsc

N = 100000
E = 1600000
F_IN = 128
H = 16

NC = 2
NS = 16
NW = NC * NS

CHUNK = 128
NCHT = 12800
EPAD = NCHT * CHUNK
NCH = NCHT // NS
SEG = 200
NSTG = NCH // SEG
SEGE = SEG * CHUNK
DUMP = SEGE + CHUNK

NPAD = 102400
HALF = NPAD // 2
ACC = HALF + 128
TNR = HALF // NS
ROUNDS = 5
RR = NPAD // (NC * ROUNDS)
QACC = RR + 128
QNR = RR // NS
CZ = 800


TRASH_W = 128


def _remap(dst_v, nrows, base, size, trash):
    lane = lax.iota(jnp.int32, 16)

    def row(j, carry):
        for k in range(CHUNK // 16):
            d = dst_v[j, pl.ds(16 * k, 16)]
            loc = d - base
            ok = (loc >= 0) & (loc < size)
            dst_v[j, pl.ds(16 * k, 16)] = jnp.where(
                ok, loc, trash + 16 * k + lane)
        return carry

    lax.fori_loop(0, nrows, row, 0)


def _deg_body(dst_hbm, out_hbm, dst_v, ones_v, stage_v, sh_deg, sem):
    c = lax.axis_index("c")
    s = lax.axis_index("s")
    base = c * HALF
    pltpu.sync_copy(dst_hbm.at[pl.ds(s * NCH, NCH)], dst_v)
    for i in range(CHUNK // 16):
        ones_v[pl.ds(16 * i, 16)] = jnp.ones((16,), jnp.float32)

    def zrow(i, carry):
        stage_v[pl.ds(16 * i, 16)] = jnp.zeros((16,), jnp.float32)
        return carry

    lax.fori_loop(0, TNR // 16, zrow, 0)
    _remap(dst_v, NCH, base, HALF, HALF)
    pltpu.sync_copy(stage_v, sh_deg.at[pl.ds(s * TNR, TNR)])
    plsc.subcore_barrier()

    def grp(p, carry):
        for b in range(8):
            pltpu.async_copy(ones_v, sh_deg.at[dst_v.at[8 * p + b]], sem, add=True)
        for b in range(8):
            pltpu.make_async_copy(ones_v, sh_deg.at[dst_v.at[8 * p + b]], sem).wait()
        return carry

    lax.fori_loop(0, NCH // 8, grp, 0)
    plsc.subcore_barrier()
    pltpu.sync_copy(sh_deg.at[pl.ds(s * TNR, TNR)], stage_v)
    pltpu.sync_copy(stage_v, out_hbm.at[pl.ds(base + s * TNR, TNR)])


def _sc_degree(dst2d):
    mesh = plsc.VectorSubcoreMesh(core_axis_name="c", subcore_axis_name="s")
    return pl.kernel(
        _deg_body,
        out_type=jax.ShapeDtypeStruct((NPAD,), jnp.float32),
        mesh=mesh,
        scratch_types=[
            pltpu.VMEM((NCH, CHUNK), jnp.int32),
            pltpu.VMEM((CHUNK,), jnp.float32),
            pltpu.VMEM((TNR,), jnp.float32),
            pltpu.VMEM_SHARED((ACC,), jnp.float32),
            pltpu.SemaphoreType.DMA,
        ],
        compiler_params=pltpu.CompilerParams(
            use_tc_tiling_on_sc=False, internal_scratch_in_bytes=65536),
    )(dst2d)


def _agg_body(g_hbm, src_hbm, dst_hbm, out_hbm,
              src_v, dst_v, rows_v, stage_v, sh_acc, semg, sems):
    c = lax.axis_index("c")
    s = lax.axis_index("s")

    def zrow(i, carry):
        stage_v[i] = jnp.zeros((H,), jnp.float32)
        return carry

    for r in range(ROUNDS):
        base = (r * NC + c) * RR
        lax.fori_loop(0, QNR, zrow, 0)
        pltpu.sync_copy(stage_v, sh_acc.at[pl.ds(s * QNR, QNR)])
        plsc.subcore_barrier()

        for g in range(NSTG):
            cb = s * NCH + g * SEG
            pltpu.sync_copy(src_hbm.at[pl.ds(cb, SEG)], src_v)
            pltpu.sync_copy(dst_hbm.at[pl.ds(cb, SEG)], dst_v)
            _remap(dst_v, SEG, base, RR, RR)

            nblk = SEG // 8

            def fire_gathers(b):
                h = 8 * lax.rem(b, 2)
                for q in range(8):
                    pltpu.async_copy(g_hbm.at[src_v.at[8 * b + q]],
                                     rows_v.at[h + q], semg)

            fire_gathers(jnp.int32(0))

            def blk(b, carry):
                h = 8 * lax.rem(b, 2)

                @pl.when(b < nblk - 1)
                def _():
                    fire_gathers(b + 1)

                for q in range(8):
                    pltpu.make_async_copy(g_hbm.at[src_v.at[8 * b + q]],
                                          rows_v.at[h + q], semg).wait()
                    pltpu.async_copy(rows_v.at[h + q],
                                     sh_acc.at[dst_v.at[8 * b + q]], sems,
                                     add=True)
                for q in range(8):
                    pltpu.make_async_copy(rows_v.at[h + q],
                                          sh_acc.at[dst_v.at[8 * b + q]],
                                          sems).wait()
                return carry

            lax.fori_loop(0, nblk, blk, 0)

        plsc.subcore_barrier()
        pltpu.sync_copy(sh_acc.at[pl.ds(s * QNR, QNR)], stage_v)
        pltpu.sync_copy(stage_v, out_hbm.at[pl.ds(base + s * QNR, QNR)])
        plsc.subcore_barrier()


def _sc_aggregate(g, src2d, dst2d):
    mesh = plsc.VectorSubcoreMesh(core_axis_name="c", subcore_axis_name="s")
    return pl.kernel(
        _agg_body,
        out_type=jax.ShapeDtypeStruct((NPAD, H), jnp.float32),
        mesh=mesh,
        scratch_types=[
            pltpu.VMEM((SEG, CHUNK), jnp.int32),
            pltpu.VMEM((SEG, CHUNK), jnp.int32),
            pltpu.VMEM((16, CHUNK, H), jnp.float32),
            pltpu.VMEM((QNR, H), jnp.float32),
            pltpu.VMEM_SHARED((QACC, H), jnp.float32),
            pltpu.SemaphoreType.DMA,
            pltpu.SemaphoreType.DMA,
        ],
        compiler_params=pltpu.CompilerParams(
            use_tc_tiling_on_sc=False, internal_scratch_in_bytes=65536),
    )(g, src2d, dst2d)


R = 2048
GRID = (N + R - 1) // R


def _tc1_body(x_ref, w1_ref, d_ref, g_ref, dis_ref):
    deg = d_ref[...] + 1.0
    dis = lax.rsqrt(deg)
    h = jnp.dot(x_ref[...], w1_ref[...], preferred_element_type=jnp.float32)
    g_ref[...] = h * dis[:, None]
    dis_ref[...] = dis


def _tc1(x, W1, deg):
    return pl.pallas_call(
        _tc1_body,
        grid=(GRID,),
        in_specs=[
            pl.BlockSpec((R, F_IN), lambda j: (j, 0)),
            pl.BlockSpec((F_IN, H), lambda j: (0, 0)),
            pl.BlockSpec((R,), lambda j: (j,)),
        ],
        out_specs=[
            pl.BlockSpec((R, H), lambda j: (j, 0)),
            pl.BlockSpec((R,), lambda j: (j,)),
        ],
        out_shape=[
            jax.ShapeDtypeStruct((N, H), jnp.float32),
            jax.ShapeDtypeStruct((N,), jnp.float32),
        ],
    )(x, W1, deg)


def _tc2_body(s_ref, g1_ref, dis_ref, b1_ref, w2_ref, g2_ref):
    dis = dis_ref[...][:, None]
    ssum = s_ref[...] + g1_ref[...]
    h2 = jnp.maximum(dis * ssum + b1_ref[...][None, :], 0.0)
    g2_ref[...] = jnp.dot(h2, w2_ref[...], preferred_element_type=jnp.float32) * dis


def _tc2(sagg, g1, dis, b1, W2):
    return pl.pallas_call(
        _tc2_body,
        grid=(GRID,),
        in_specs=[
            pl.BlockSpec((R, H), lambda j: (j, 0)),
            pl.BlockSpec((R, H), lambda j: (j, 0)),
            pl.BlockSpec((R,), lambda j: (j,)),
            pl.BlockSpec((H,), lambda j: (0,)),
            pl.BlockSpec((H, H), lambda j: (0, 0)),
        ],
        out_specs=pl.BlockSpec((R, H), lambda j: (j, 0)),
        out_shape=jax.ShapeDtypeStruct((N, H), jnp.float32),
    )(sagg, g1, dis, b1, W2)


def _tc3_body(s_ref, g2_ref, dis_ref, b2_ref, wf_ref, bf_ref, out_ref):
    dis = dis_ref[...][:, None]
    ssum = s_ref[...] + g2_ref[...]
    h3 = jnp.maximum(dis * ssum + b2_ref[...][None, :], 0.0)
    out_ref[...] = (jnp.dot(h3, wf_ref[...], preferred_element_type=jnp.float32)
                    + bf_ref[...][None, :])


def _tc3(sagg, g2, dis, b2, Wf, bf):
    return pl.pallas_call(
        _tc3_body,
        grid=(GRID,),
        in_specs=[
            pl.BlockSpec((R, H), lambda j: (j, 0)),
            pl.BlockSpec((R, H), lambda j: (j, 0)),
            pl.BlockSpec((R,), lambda j: (j,)),
            pl.BlockSpec((H,), lambda j: (0,)),
            pl.BlockSpec((H, H), lambda j: (0, 0)),
            pl.BlockSpec((H,), lambda j: (0,)),
        ],
        out_specs=pl.BlockSpec((R, H), lambda j: (j, 0)),
        out_shape=jax.ShapeDtypeStruct((N, H), jnp.float32),
    )(sagg, g2, dis, b2, Wf, bf)


def kernel(x, edge_index, W1, b1, W2, b2, Wf, bf):
    src = edge_index[0]
    dst = edge_index[1]
    npad_e = EPAD - E
    src2d = jnp.concatenate(
        [src, jnp.zeros((npad_e,), jnp.int32)]).reshape(NCHT, CHUNK)
    dst2d = jnp.concatenate(
        [dst, jnp.full((npad_e,), N, jnp.int32)]).reshape(NCHT, CHUNK)

    deg = _sc_degree(dst2d)
    g1, dis = _tc1(x, W1, deg)
    s1 = _sc_aggregate(g1, src2d, dst2d)
    g2 = _tc2(s1, g1, dis, b1, W2)
    s2 = _sc_aggregate(g2, src2d, dst2d)
    return _tc3(s2, g2, dis, b2, Wf, bf)
